# baseline (speedup 1.0000x reference)
import functools

import jax
import jax.numpy as jnp
import numpy as np
from jax import lax
from jax.experimental import pallas as pl
from jax.experimental.pallas import tpu as pltpu
from jax.experimental.pallas import tpu_sc as plsc

_N = 10000
_D = 128
_E = 320000
_NC = 2
_NS = 16
_NW = _NC * _NS
_BATCH = 64
_NB0 = 160
_NB1 = 153
_NBMAX = max(_NB0, _NB1)
_ROWS_PAD = _NS * _NB0 + (_NS - 1) * _NB1 + _NBMAX
_NP = 10112
_ZROWS = _NP // _NS


def _sc_agg(table, srcs, dsts, zeros, compute_deg):
  mesh = plsc.VectorSubcoreMesh(core_axis_name="c", subcore_axis_name="s")
  out_type = [jax.ShapeDtypeStruct((_NC, _NP, _D), jnp.float32)]
  if compute_deg:
    out_type.append(jax.ShapeDtypeStruct((_NW, 2, _BATCH, _D), jnp.float32))
  nbuf = 3
  scratch = [
      pltpu.VMEM_SHARED((_NP, _D), jnp.float32),
      pltpu.VMEM((_NBMAX, _BATCH), jnp.int32),
      pltpu.VMEM((_NBMAX, _BATCH), jnp.int32),
      pltpu.VMEM((nbuf, _BATCH, _D), jnp.float32),
  ] + [pltpu.SemaphoreType.DMA] * (2 * nbuf)

  def body(table_ref, src_ref, dst_ref, zero_ref, *rest):
    if compute_deg:
      part_ref, degp_ref, acc, src_v, dst_v, rows, *sems = rest
    else:
      part_ref, acc, src_v, dst_v, rows, *sems = rest
    gsem, ssem = sems[:nbuf], sems[nbuf:]
    core = lax.axis_index("c")
    sub = lax.axis_index("s")
    wid = core * _NS + sub
    off = jnp.where(core == 0, sub * _NB0, _NS * _NB0 + sub * _NB1)
    ih0 = pltpu.async_copy(src_ref.at[pl.ds(off, _NBMAX)], src_v, gsem[0])
    ih1 = pltpu.async_copy(dst_ref.at[pl.ds(off, _NBMAX)], dst_v, gsem[1])
    pltpu.sync_copy(zero_ref, acc.at[pl.ds(sub * _ZROWS, _ZROWS)])
    ih0.wait()
    ih1.wait()

    if compute_deg:
      zf = jnp.zeros((16,), jnp.float32)
      def zstep(i, c):
        k = i * 16
        rows[k // (_BATCH * _D), (k // _D) % _BATCH, pl.ds(k % _D, 16)] = zf
        return c
      lax.fori_loop(0, 2 * _BATCH * _D // 16, zstep, 0)
      onef = jnp.ones((16,), jnp.float32)
      g = _BATCH // 16
      nb_me = jnp.where(core == 0, _NB0, _NB1)
      def dstep(i, c):
        d = dst_v[i // g, pl.ds((i % g) * 16, 16)]
        i0 = lax.shift_right_logical(d, _BATCH.bit_length() - 1 + 7)
        i1 = jnp.bitwise_and(lax.shift_right_logical(d, 7), _BATCH - 1)
        i2 = jnp.bitwise_and(d, _D - 1)
        plsc.addupdate_scatter(rows, [i0, i1, i2], onef)
        return c
      lax.fori_loop(0, nb_me * g, dstep, 0)
      pltpu.sync_copy(rows.at[pl.ds(0, 2)], degp_ref.at[wid])

    plsc.subcore_barrier()

    def gather(m):
      s = m % nbuf
      return pltpu.async_copy(table_ref.at[src_v.at[m]], rows.at[s], gsem[s])

    def pipeline(nb):
      gh = [None] * nbuf
      sh = [None] * nbuf
      gh[0] = gather(0)
      gh[1] = gather(1)
      for j in range(nb):
        s = j % 3
        w = (j + 2) % 3
        if j + 2 < nb:
          if sh[w] is not None:
            sh[w].wait()
          gh[w] = gather(j + 2)
        gh[s].wait()
        sh[s] = pltpu.async_copy(rows.at[s], acc.at[dst_v.at[j]], ssem[s],
                                 add=True)
      for t in range(min(3, nb)):
        h = sh[(nb - 1 - t) % 3]
        if h is not None:
          h.wait()

    @pl.when(core == 0)
    def _():
      pipeline(_NB0)

    @pl.when(core == 1)
    def _():
      pipeline(_NB1)

    plsc.subcore_barrier()
    pltpu.sync_copy(acc.at[pl.ds(sub * _ZROWS, _ZROWS)],
                    part_ref.at[core, pl.ds(sub * _ZROWS, _ZROWS)])

  fn = pl.kernel(body, out_type=tuple(out_type), mesh=mesh,
                 scratch_types=tuple(scratch),
                 compiler_params=pltpu.CompilerParams(
                     needs_layout_passes=False,
                     use_tc_tiling_on_sc=False))
  return fn(table, srcs, dsts, zeros)


def _combine1(parts, degp_t, x, wl, wr, b):
  def body(p_ref, degp_ref, x_ref, wl_ref, wr_ref, b_ref, h_ref, degc_ref):
    deg = jnp.sum(degp_ref[...], axis=1, keepdims=True)
    degc = jnp.maximum(deg, 1.0)
    s = p_ref[0, :_N, :] + p_ref[1, :_N, :]
    h = jnp.dot(s / degc, wl_ref[...], preferred_element_type=jnp.float32)
    h = h + b_ref[...] + jnp.dot(x_ref[...], wr_ref[...],
                                 preferred_element_type=jnp.float32)
    h_ref[...] = jnp.maximum(h, 0.0)
    degc_ref[...] = degc
  return pl.pallas_call(
      body,
      out_shape=(jax.ShapeDtypeStruct((_N, _D), jnp.float32),
                 jax.ShapeDtypeStruct((_N, 1), jnp.float32)),
  )(parts, degp_t, x, wl, wr, b)


def _combine2(parts, degc, hprev, wl, wr, b):
  def body(p_ref, degc_ref, hp_ref, wl_ref, wr_ref, b_ref, h_ref):
    s = p_ref[0, :_N, :] + p_ref[1, :_N, :]
    h = jnp.dot(s / degc_ref[...], wl_ref[...],
                preferred_element_type=jnp.float32)
    h = h + b_ref[...] + jnp.dot(hp_ref[...], wr_ref[...],
                                 preferred_element_type=jnp.float32)
    h_ref[...] = jnp.maximum(h, 0.0)
  return pl.pallas_call(
      body,
      out_shape=jax.ShapeDtypeStruct((_N, _D), jnp.float32),
  )(parts, degc, hprev, wl, wr, b)


def _combine3(parts, degc, h2, wl, wr, b, h1):
  def body(p_ref, degc_ref, h2_ref, wl_ref, wr_ref, b_ref, h1_ref, o_ref):
    s = p_ref[0, :_N, :] + p_ref[1, :_N, :]
    h3 = jnp.dot(s / degc_ref[...], wl_ref[...],
                 preferred_element_type=jnp.float32)
    h3 = h3 + b_ref[...] + jnp.dot(h2_ref[...], wr_ref[...],
                                   preferred_element_type=jnp.float32)
    o_ref[...] = jnp.maximum(jnp.maximum(h1_ref[...], h2_ref[...]), h3)
  return pl.pallas_call(
      body,
      out_shape=jax.ShapeDtypeStruct((_N, _D), jnp.float32),
  )(parts, degc, h2, wl, wr, b, h1)


_EPAD = _ROWS_PAD * _BATCH


def kernel(x, adj_t, W1l, b1, W1r, W2l, b2, W2r, W3l, b3, W3r):
  src = adj_t[0]
  dst = adj_t[1]
  srcs = jnp.concatenate([src, jnp.zeros((_EPAD - _E,), jnp.int32)])
  srcs = srcs.reshape(_ROWS_PAD, _BATCH)
  dsts = jnp.concatenate([dst, jnp.full((_EPAD - _E,), _N, jnp.int32)])
  dsts = dsts.reshape(_ROWS_PAD, _BATCH)
  zeros = jnp.zeros((_ZROWS, _D), jnp.float32)
  b1r, b2r, b3r = b1.reshape(1, _D), b2.reshape(1, _D), b3.reshape(1, _D)

  p1, degp = _sc_agg(x, srcs, dsts, zeros, True)
  degp_t = degp.reshape(_NW, 2 * _BATCH * _D)[:, :_N].T
  h1, degc = _combine1(p1, degp_t, x, W1l, W1r, b1r)
  (p2,) = _sc_agg(h1, srcs, dsts, zeros, False)
  h2 = _combine2(p2, degc, h1, W2l, W2r, b2r)
  (p3,) = _sc_agg(h2, srcs, dsts, zeros, False)
  return _combine3(p3, degc, h2, W3l, W3r, b3r, h1)


# speedup vs baseline: 1.0343x; 1.0343x over previous
import functools

import jax
import jax.numpy as jnp
import numpy as np
from jax import lax
from jax.experimental import pallas as pl
from jax.experimental.pallas import tpu as pltpu
from jax.experimental.pallas import tpu_sc as plsc

_N = 10000
_D = 128
_E = 320000
_NC = 2
_NS = 16
_NW = _NC * _NS
_BATCH = 64
_NB0 = 172
_NB1 = 141
_NBMAX = max(_NB0, _NB1)
_ROWS_PAD = _NS * _NB0 + (_NS - 1) * _NB1 + _NBMAX
_NP = 10112
_ZROWS = _NP // _NS


def _sc_agg(table, srcs, dsts, zeros, compute_deg):
  mesh = plsc.VectorSubcoreMesh(core_axis_name="c", subcore_axis_name="s")
  out_type = [jax.ShapeDtypeStruct((_NC, _NP, _D), jnp.float32)]
  if compute_deg:
    out_type.append(jax.ShapeDtypeStruct((_NW, 2, _BATCH, _D), jnp.float32))
  nbuf = 3
  scratch = [
      pltpu.VMEM_SHARED((_NP, _D), jnp.float32),
      pltpu.VMEM((_NBMAX, _BATCH), jnp.int32),
      pltpu.VMEM((_NBMAX, _BATCH), jnp.int32),
      pltpu.VMEM((nbuf, _BATCH, _D), jnp.float32),
  ] + [pltpu.SemaphoreType.DMA] * (2 * nbuf)

  def body(table_ref, src_ref, dst_ref, zero_ref, *rest):
    if compute_deg:
      part_ref, degp_ref, acc, src_v, dst_v, rows, *sems = rest
    else:
      part_ref, acc, src_v, dst_v, rows, *sems = rest
    gsem, ssem = sems[:nbuf], sems[nbuf:]
    core = lax.axis_index("c")
    sub = lax.axis_index("s")
    wid = core * _NS + sub
    off = jnp.where(core == 0, sub * _NB0, _NS * _NB0 + sub * _NB1)
    ih0 = pltpu.async_copy(src_ref.at[pl.ds(off, _NBMAX)], src_v, gsem[0])
    ih1 = pltpu.async_copy(dst_ref.at[pl.ds(off, _NBMAX)], dst_v, gsem[1])
    pltpu.sync_copy(zero_ref, acc.at[pl.ds(sub * _ZROWS, _ZROWS)])
    ih0.wait()
    ih1.wait()

    if compute_deg:
      zf = jnp.zeros((16,), jnp.float32)
      def zstep(i, c):
        k = i * 16
        rows[k // (_BATCH * _D), (k // _D) % _BATCH, pl.ds(k % _D, 16)] = zf
        return c
      lax.fori_loop(0, 2 * _BATCH * _D // 16, zstep, 0)
      onef = jnp.ones((16,), jnp.float32)
      g = _BATCH // 16
      nb_me = jnp.where(core == 0, _NB0, _NB1)
      def dstep(i, c):
        d = dst_v[i // g, pl.ds((i % g) * 16, 16)]
        i0 = lax.shift_right_logical(d, _BATCH.bit_length() - 1 + 7)
        i1 = jnp.bitwise_and(lax.shift_right_logical(d, 7), _BATCH - 1)
        i2 = jnp.bitwise_and(d, _D - 1)
        plsc.addupdate_scatter(rows, [i0, i1, i2], onef)
        return c
      lax.fori_loop(0, nb_me * g, dstep, 0)
      pltpu.sync_copy(rows.at[pl.ds(0, 2)], degp_ref.at[wid])

    plsc.subcore_barrier()

    def gather(m):
      s = m % nbuf
      return pltpu.async_copy(table_ref.at[src_v.at[m]], rows.at[s], gsem[s])

    def pipeline(nb):
      gh = [None] * nbuf
      sh = [None] * nbuf
      gh[0] = gather(0)
      gh[1] = gather(1)
      for j in range(nb):
        s = j % 3
        w = (j + 2) % 3
        if j + 2 < nb:
          if sh[w] is not None:
            sh[w].wait()
          gh[w] = gather(j + 2)
        gh[s].wait()
        sh[s] = pltpu.async_copy(rows.at[s], acc.at[dst_v.at[j]], ssem[s],
                                 add=True)
      for t in range(min(3, nb)):
        h = sh[(nb - 1 - t) % 3]
        if h is not None:
          h.wait()

    @pl.when(core == 0)
    def _():
      pipeline(_NB0)

    @pl.when(core == 1)
    def _():
      pipeline(_NB1)

    plsc.subcore_barrier()
    pltpu.sync_copy(acc.at[pl.ds(sub * _ZROWS, _ZROWS)],
                    part_ref.at[core, pl.ds(sub * _ZROWS, _ZROWS)])

  fn = pl.kernel(body, out_type=tuple(out_type), mesh=mesh,
                 scratch_types=tuple(scratch),
                 compiler_params=pltpu.CompilerParams(
                     needs_layout_passes=False,
                     use_tc_tiling_on_sc=False))
  return fn(table, srcs, dsts, zeros)


def _combine1(parts, degp_t, x, wl, wr, b):
  def body(p_ref, degp_ref, x_ref, wl_ref, wr_ref, b_ref, h_ref, degc_ref):
    deg = jnp.sum(degp_ref[...], axis=1, keepdims=True)
    degc = jnp.maximum(deg, 1.0)
    s = p_ref[0, :_N, :] + p_ref[1, :_N, :]
    h = jnp.dot(s / degc, wl_ref[...], preferred_element_type=jnp.float32)
    h = h + b_ref[...] + jnp.dot(x_ref[...], wr_ref[...],
                                 preferred_element_type=jnp.float32)
    h_ref[...] = jnp.maximum(h, 0.0)
    degc_ref[...] = degc
  return pl.pallas_call(
      body,
      out_shape=(jax.ShapeDtypeStruct((_N, _D), jnp.float32),
                 jax.ShapeDtypeStruct((_N, 1), jnp.float32)),
  )(parts, degp_t, x, wl, wr, b)


def _combine2(parts, degc, hprev, wl, wr, b):
  def body(p_ref, degc_ref, hp_ref, wl_ref, wr_ref, b_ref, h_ref):
    s = p_ref[0, :_N, :] + p_ref[1, :_N, :]
    h = jnp.dot(s / degc_ref[...], wl_ref[...],
                preferred_element_type=jnp.float32)
    h = h + b_ref[...] + jnp.dot(hp_ref[...], wr_ref[...],
                                 preferred_element_type=jnp.float32)
    h_ref[...] = jnp.maximum(h, 0.0)
  return pl.pallas_call(
      body,
      out_shape=jax.ShapeDtypeStruct((_N, _D), jnp.float32),
  )(parts, degc, hprev, wl, wr, b)


def _combine3(parts, degc, h2, wl, wr, b, h1):
  def body(p_ref, degc_ref, h2_ref, wl_ref, wr_ref, b_ref, h1_ref, o_ref):
    s = p_ref[0, :_N, :] + p_ref[1, :_N, :]
    h3 = jnp.dot(s / degc_ref[...], wl_ref[...],
                 preferred_element_type=jnp.float32)
    h3 = h3 + b_ref[...] + jnp.dot(h2_ref[...], wr_ref[...],
                                   preferred_element_type=jnp.float32)
    o_ref[...] = jnp.maximum(jnp.maximum(h1_ref[...], h2_ref[...]), h3)
  return pl.pallas_call(
      body,
      out_shape=jax.ShapeDtypeStruct((_N, _D), jnp.float32),
  )(parts, degc, h2, wl, wr, b, h1)


_EPAD = _ROWS_PAD * _BATCH


def kernel(x, adj_t, W1l, b1, W1r, W2l, b2, W2r, W3l, b3, W3r):
  src = adj_t[0]
  dst = adj_t[1]
  srcs = jnp.concatenate([src, jnp.zeros((_EPAD - _E,), jnp.int32)])
  srcs = srcs.reshape(_ROWS_PAD, _BATCH)
  dsts = jnp.concatenate([dst, jnp.full((_EPAD - _E,), _N, jnp.int32)])
  dsts = dsts.reshape(_ROWS_PAD, _BATCH)
  zeros = jnp.zeros((_ZROWS, _D), jnp.float32)
  b1r, b2r, b3r = b1.reshape(1, _D), b2.reshape(1, _D), b3.reshape(1, _D)

  p1, degp = _sc_agg(x, srcs, dsts, zeros, True)
  degp_t = degp.reshape(_NW, 2 * _BATCH * _D)[:, :_N].T
  h1, degc = _combine1(p1, degp_t, x, W1l, W1r, b1r)
  (p2,) = _sc_agg(h1, srcs, dsts, zeros, False)
  h2 = _combine2(p2, degc, h1, W2l, W2r, b2r)
  (p3,) = _sc_agg(h2, srcs, dsts, zeros, False)
  return _combine3(p3, degc, h2, W3l, W3r, b3r, h1)
